# Initial kernel scaffold; baseline (speedup 1.0000x reference)
import functools

import jax
import jax.numpy as jnp
from jax import lax
from jax.experimental import pallas as pl
from jax.experimental.pallas import tpu as pltpu
from jax.experimental.pallas import tpu_sc as plsc

N = 10000
NP = 10240
E = 320000
NC, NS, LANES = 2, 16, 16
NW = NC * NS
CH = 128
NCH = 80
EPW = NCH * CH
EPAD = NW * EPW
RPT = NP // NS
BLK = 1024
GRID = NP // BLK

_f32 = jnp.float32


def _sc_mesh():
    return plsc.VectorSubcoreMesh(
        core_axis_name="c", subcore_axis_name="s",
        num_cores=NC, num_subcores=NS)


def _make_agg(D):
    @functools.partial(
        pl.kernel,
        out_type=jax.ShapeDtypeStruct((NC, NP, D), _f32),
        mesh=_sc_mesh(),
        scratch_types=[
            pltpu.VMEM((NCH, CH), jnp.int32),
            pltpu.VMEM((NCH, CH), jnp.int32),
            pltpu.VMEM((CH, D), _f32),
            pltpu.VMEM_SHARED((NP, D), _f32),
            pltpu.SemaphoreType.DMA,
        ],
    )
    def agg(p_hbm, src_hbm, dst_hbm, z_hbm, out_hbm,
            src_v, dst_v, rows_v, acc_sh, sem):
        c = lax.axis_index("c")
        s = lax.axis_index("s")
        wid = s * NC + c
        pltpu.sync_copy(z_hbm, acc_sh.at[pl.ds(s * RPT, RPT)])
        pltpu.sync_copy(src_hbm.at[wid], src_v)
        pltpu.sync_copy(dst_hbm.at[wid], dst_v)
        plsc.subcore_barrier()

        def body(j, carry):
            pltpu.async_copy(p_hbm.at[src_v.at[j]], rows_v, sem).wait()
            pltpu.sync_copy(rows_v, acc_sh.at[dst_v.at[j]], add=True)
            return carry

        lax.fori_loop(0, NCH, body, 0)
        plsc.subcore_barrier()
        pltpu.sync_copy(acc_sh.at[pl.ds(s * RPT, RPT)],
                        out_hbm.at[c, pl.ds(s * RPT, RPT)])

    return agg


def _make_deg():
    @functools.partial(
        pl.kernel,
        out_type=jax.ShapeDtypeStruct((NC, NP, LANES), _f32),
        mesh=_sc_mesh(),
        scratch_types=[
            pltpu.VMEM((NCH, CH), jnp.int32),
            pltpu.VMEM((CH, LANES), _f32),
            pltpu.VMEM_SHARED((NP, LANES), _f32),
        ],
    )
    def deg(dst_hbm, ones_hbm, z_hbm, out_hbm, dst_v, ones_v, acc_sh):
        c = lax.axis_index("c")
        s = lax.axis_index("s")
        wid = s * NC + c
        pltpu.sync_copy(z_hbm, acc_sh.at[pl.ds(s * RPT, RPT)])
        pltpu.sync_copy(ones_hbm, ones_v)
        pltpu.sync_copy(dst_hbm.at[wid], dst_v)
        plsc.subcore_barrier()

        def body(j, carry):
            pltpu.sync_copy(ones_v, acc_sh.at[dst_v.at[j]], add=True)
            return carry

        lax.fori_loop(0, NCH, body, 0)
        plsc.subcore_barrier()
        pltpu.sync_copy(acc_sh.at[pl.ds(s * RPT, RPT)],
                        out_hbm.at[c, pl.ds(s * RPT, RPT)])

    return deg


def _srow(deg_blk):
    d = deg_blk[0, :, 0:1] + deg_blk[1, :, 0:1] + 1.0
    return lax.rsqrt(d)


def _k_first(x_ref, deg_ref, w_ref, b_ref, o_ref):
    s = _srow(deg_ref[...])
    m = jnp.dot(x_ref[...], w_ref[...], preferred_element_type=_f32)
    o_ref[...] = s * (m + b_ref[...])


def _k_mid(a_ref, p_ref, deg_ref, w_ref, b_ref, o_ref):
    s = _srow(deg_ref[...])
    aa = a_ref[...]
    agg = s * (aa[0] + aa[1] + p_ref[...])
    h = jnp.where(agg > 0, agg, jnp.expm1(agg))
    m = jnp.dot(h, w_ref[...], preferred_element_type=_f32)
    o_ref[...] = s * (m + b_ref[...])


def _k_last(a_ref, p_ref, deg_ref, o_ref):
    s = _srow(deg_ref[...])
    aa = a_ref[...]
    z = s * (aa[0] + aa[1] + p_ref[...])
    col = lax.broadcasted_iota(jnp.int32, z.shape, 1)
    zm = jnp.where(col < 28, z, -1e30)
    mx = jnp.max(zm, axis=-1, keepdims=True)
    e = jnp.exp(zm - mx)
    o_ref[...] = e / jnp.sum(e, axis=-1, keepdims=True)


def _row_spec(d):
    return pl.BlockSpec((BLK, d), lambda i: (i, 0))


def _deg_spec():
    return pl.BlockSpec((NC, BLK, LANES), lambda i: (0, i, 0))


def _part_spec(d):
    return pl.BlockSpec((NC, BLK, d), lambda i: (0, i, 0))


def _full_spec(shape):
    return pl.BlockSpec(shape, lambda i: tuple(0 for _ in shape))


def _tc_first(x, deg, w, b):
    din, dout = w.shape
    return pl.pallas_call(
        _k_first,
        grid=(GRID,),
        in_specs=[_row_spec(din), _deg_spec(),
                  _full_spec((din, dout)), _full_spec((1, dout))],
        out_specs=_row_spec(dout),
        out_shape=jax.ShapeDtypeStruct((NP, dout), _f32),
    )(x, deg, w, b)


def _tc_mid(a, p, deg, w, b):
    din, dout = w.shape
    return pl.pallas_call(
        _k_mid,
        grid=(GRID,),
        in_specs=[_part_spec(din), _row_spec(din), _deg_spec(),
                  _full_spec((din, dout)), _full_spec((1, dout))],
        out_specs=_row_spec(dout),
        out_shape=jax.ShapeDtypeStruct((NP, dout), _f32),
    )(a, p, deg, w, b)


def _tc_last(a, p, deg):
    d = p.shape[1]
    return pl.pallas_call(
        _k_last,
        grid=(GRID,),
        in_specs=[_part_spec(d), _row_spec(d), _deg_spec()],
        out_specs=_row_spec(d),
        out_shape=jax.ShapeDtypeStruct((NP, d), _f32),
    )(a, p, deg)


def kernel(x, edge_index, W1, b1, W2, b2, W3, b3, W4, b4):
    ei = edge_index.astype(jnp.int32)
    pad = jnp.full((EPAD - E,), N, jnp.int32)
    src_r = jnp.concatenate([ei[0], pad]).reshape(NW, NCH, CH)
    dst_r = jnp.concatenate([ei[1], pad]).reshape(NW, NCH, CH)
    xp = jnp.pad(x, ((0, NP - N), (0, 0)))
    w4p = jnp.pad(W4, ((0, 0), (0, 4)))
    b4p = jnp.pad(b4, (0, 4))

    ones16 = jnp.ones((CH, LANES), _f32)
    z16 = jnp.zeros((RPT, LANES), _f32)
    z32 = jnp.zeros((RPT, 32), _f32)
    z64 = jnp.zeros((RPT, 64), _f32)

    deg = _make_deg()(dst_r, ones16, z16)
    agg32 = _make_agg(32)
    agg64 = _make_agg(64)

    p1 = _tc_first(xp, deg, W1, b1.reshape(1, -1))
    a1 = agg32(p1, src_r, dst_r, z32)
    p2 = _tc_mid(a1, p1, deg, W2, b2.reshape(1, -1))
    a2 = agg64(p2, src_r, dst_r, z64)
    p3 = _tc_mid(a2, p2, deg, W3, b3.reshape(1, -1))
    a3 = agg32(p3, src_r, dst_r, z32)
    p4 = _tc_mid(a3, p3, deg, w4p, b4p.reshape(1, -1))
    a4 = agg32(p4, src_r, dst_r, z32)
    out = _tc_last(a4, p4, deg)
    return out[:N, :28]


# speedup vs baseline: 16.7470x; 16.7470x over previous
import functools

import jax
import jax.numpy as jnp
from jax import lax
from jax.experimental import pallas as pl
from jax.experimental.pallas import tpu as pltpu
from jax.experimental.pallas import tpu_sc as plsc

N = 10000
NP = 10240
E = 320000
NC, NS, LANES = 2, 16, 16
NW = NC * NS
CH = 128
NCH = 80
EPW = NCH * CH
EPAD = NW * EPW
RPT = NP // NS
BLK = 1024
GRID = NP // BLK

_f32 = jnp.float32


def _sc_mesh():
    return plsc.VectorSubcoreMesh(
        core_axis_name="c", subcore_axis_name="s",
        num_cores=NC, num_subcores=NS)


def _make_agg(D):
    @functools.partial(
        pl.kernel,
        out_type=jax.ShapeDtypeStruct((NC, NP, D), _f32),
        mesh=_sc_mesh(),
        compiler_params=pltpu.CompilerParams(use_tc_tiling_on_sc=False),
        scratch_types=[
            pltpu.VMEM((NCH, CH), jnp.int32),
            pltpu.VMEM((NCH, CH), jnp.int32),
            pltpu.VMEM((CH, D), _f32),
            pltpu.VMEM_SHARED((NP, D), _f32),
            pltpu.SemaphoreType.DMA,
        ],
    )
    def agg(p_hbm, src_hbm, dst_hbm, z_hbm, out_hbm,
            src_v, dst_v, rows_v, acc_sh, sem):
        c = lax.axis_index("c")
        s = lax.axis_index("s")
        wid = s * NC + c
        pltpu.sync_copy(z_hbm, acc_sh.at[pl.ds(s * RPT, RPT)])
        pltpu.sync_copy(src_hbm.at[wid], src_v)
        pltpu.sync_copy(dst_hbm.at[wid], dst_v)
        plsc.subcore_barrier()

        def body(j, carry):
            pltpu.async_copy(p_hbm.at[src_v.at[j]], rows_v, sem).wait()
            pltpu.sync_copy(rows_v, acc_sh.at[dst_v.at[j]], add=True)
            return carry

        lax.fori_loop(0, NCH, body, 0)
        plsc.subcore_barrier()
        pltpu.sync_copy(acc_sh.at[pl.ds(s * RPT, RPT)],
                        out_hbm.at[c, pl.ds(s * RPT, RPT)])

    return agg


def _make_deg():
    @functools.partial(
        pl.kernel,
        out_type=jax.ShapeDtypeStruct((NC, NP, LANES), _f32),
        mesh=_sc_mesh(),
        compiler_params=pltpu.CompilerParams(use_tc_tiling_on_sc=False),
        scratch_types=[
            pltpu.VMEM((NCH, CH), jnp.int32),
            pltpu.VMEM((CH, LANES), _f32),
            pltpu.VMEM_SHARED((NP, LANES), _f32),
        ],
    )
    def deg(dst_hbm, ones_hbm, z_hbm, out_hbm, dst_v, ones_v, acc_sh):
        c = lax.axis_index("c")
        s = lax.axis_index("s")
        wid = s * NC + c
        pltpu.sync_copy(z_hbm, acc_sh.at[pl.ds(s * RPT, RPT)])
        pltpu.sync_copy(ones_hbm, ones_v)
        pltpu.sync_copy(dst_hbm.at[wid], dst_v)
        plsc.subcore_barrier()

        def body(j, carry):
            pltpu.sync_copy(ones_v, acc_sh.at[dst_v.at[j]], add=True)
            return carry

        lax.fori_loop(0, NCH, body, 0)
        plsc.subcore_barrier()
        pltpu.sync_copy(acc_sh.at[pl.ds(s * RPT, RPT)],
                        out_hbm.at[c, pl.ds(s * RPT, RPT)])

    return deg


def _srow(deg_blk):
    d = deg_blk[0, :, 0:1] + deg_blk[1, :, 0:1] + 1.0
    return lax.rsqrt(d)


def _k_first(x_ref, deg_ref, w_ref, b_ref, o_ref):
    s = _srow(deg_ref[...])
    m = jnp.dot(x_ref[...], w_ref[...], preferred_element_type=_f32)
    o_ref[...] = s * (m + b_ref[...])


def _k_mid(a_ref, p_ref, deg_ref, w_ref, b_ref, o_ref):
    s = _srow(deg_ref[...])
    aa = a_ref[...]
    agg = s * (aa[0] + aa[1] + p_ref[...])
    h = jnp.where(agg > 0, agg, jnp.exp(agg) - 1.0)
    m = jnp.dot(h, w_ref[...], preferred_element_type=_f32)
    o_ref[...] = s * (m + b_ref[...])


def _k_last(a_ref, p_ref, deg_ref, o_ref):
    s = _srow(deg_ref[...])
    aa = a_ref[...]
    z = s * (aa[0] + aa[1] + p_ref[...])
    col = lax.broadcasted_iota(jnp.int32, z.shape, 1)
    zm = jnp.where(col < 28, z, -1e30)
    mx = jnp.max(zm, axis=-1, keepdims=True)
    e = jnp.exp(zm - mx)
    o_ref[...] = e / jnp.sum(e, axis=-1, keepdims=True)


def _row_spec(d):
    return pl.BlockSpec((BLK, d), lambda i: (i, 0))


def _deg_spec():
    return pl.BlockSpec((NC, BLK, LANES), lambda i: (0, i, 0))


def _part_spec(d):
    return pl.BlockSpec((NC, BLK, d), lambda i: (0, i, 0))


def _full_spec(shape):
    return pl.BlockSpec(shape, lambda i: tuple(0 for _ in shape))


def _tc_first(x, deg, w, b):
    din, dout = w.shape
    return pl.pallas_call(
        _k_first,
        grid=(GRID,),
        in_specs=[_row_spec(din), _deg_spec(),
                  _full_spec((din, dout)), _full_spec((1, dout))],
        out_specs=_row_spec(dout),
        out_shape=jax.ShapeDtypeStruct((NP, dout), _f32),
    )(x, deg, w, b)


def _tc_mid(a, p, deg, w, b):
    din, dout = w.shape
    return pl.pallas_call(
        _k_mid,
        grid=(GRID,),
        in_specs=[_part_spec(din), _row_spec(din), _deg_spec(),
                  _full_spec((din, dout)), _full_spec((1, dout))],
        out_specs=_row_spec(dout),
        out_shape=jax.ShapeDtypeStruct((NP, dout), _f32),
    )(a, p, deg, w, b)


def _tc_last(a, p, deg):
    d = p.shape[1]
    return pl.pallas_call(
        _k_last,
        grid=(GRID,),
        in_specs=[_part_spec(d), _row_spec(d), _deg_spec()],
        out_specs=_row_spec(d),
        out_shape=jax.ShapeDtypeStruct((NP, d), _f32),
    )(a, p, deg)


def kernel(x, edge_index, W1, b1, W2, b2, W3, b3, W4, b4):
    ei = edge_index.astype(jnp.int32)
    pad = jnp.full((EPAD - E,), N, jnp.int32)
    src_r = jnp.concatenate([ei[0], pad]).reshape(NW, NCH, CH)
    dst_r = jnp.concatenate([ei[1], pad]).reshape(NW, NCH, CH)
    xp = jnp.pad(x, ((0, NP - N), (0, 0)))
    w4p = jnp.pad(W4, ((0, 0), (0, 4)))
    b4p = jnp.pad(b4, (0, 4))

    ones16 = jnp.ones((CH, LANES), _f32)
    z16 = jnp.zeros((RPT, LANES), _f32)
    z32 = jnp.zeros((RPT, 32), _f32)
    z64 = jnp.zeros((RPT, 64), _f32)

    deg = _make_deg()(dst_r, ones16, z16)
    agg32 = _make_agg(32)
    agg64 = _make_agg(64)

    p1 = _tc_first(xp, deg, W1, b1.reshape(1, -1))
    a1 = agg32(p1, src_r, dst_r, z32)
    p2 = _tc_mid(a1, p1, deg, W2, b2.reshape(1, -1))
    a2 = agg64(p2, src_r, dst_r, z64)
    p3 = _tc_mid(a2, p2, deg, W3, b3.reshape(1, -1))
    a3 = agg32(p3, src_r, dst_r, z32)
    p4 = _tc_mid(a3, p3, deg, w4p, b4p.reshape(1, -1))
    a4 = agg32(p4, src_r, dst_r, z32)
    out = _tc_last(a4, p4, deg)
    return out[:N, :28]


# speedup vs baseline: 20.9250x; 1.2495x over previous
import functools

import jax
import jax.numpy as jnp
from jax import lax
from jax.experimental import pallas as pl
from jax.experimental.pallas import tpu as pltpu
from jax.experimental.pallas import tpu_sc as plsc

N = 10000
NP = 10240
E = 320000
NC, NS, LANES = 2, 16, 16
NW = NC * NS
CH = 128
NCH = 80
EPW = NCH * CH
EPAD = NW * EPW
RPT = NP // NS
NBUF = 4
BLK = 1024
GRID = NP // BLK

_f32 = jnp.float32


def _sc_mesh():
    return plsc.VectorSubcoreMesh(
        core_axis_name="c", subcore_axis_name="s",
        num_cores=NC, num_subcores=NS)


def _make_agg(D):
    @functools.partial(
        pl.kernel,
        out_type=jax.ShapeDtypeStruct((NC, NP, D), _f32),
        mesh=_sc_mesh(),
        compiler_params=pltpu.CompilerParams(use_tc_tiling_on_sc=False),
        scratch_types=[
            pltpu.VMEM((NCH, CH), jnp.int32),
            pltpu.VMEM((NCH, CH), jnp.int32),
            pltpu.VMEM((NBUF, CH, D), _f32),
            pltpu.VMEM_SHARED((NP, D), _f32),
            [pltpu.SemaphoreType.DMA] * NBUF,
            [pltpu.SemaphoreType.DMA] * NBUF,
        ],
    )
    def agg(p_hbm, src_hbm, dst_hbm, z_hbm, out_hbm,
            src_v, dst_v, rows_v, acc_sh, gsems, ssems):
        c = lax.axis_index("c")
        s = lax.axis_index("s")
        wid = s * NC + c
        pltpu.sync_copy(z_hbm, acc_sh.at[pl.ds(s * RPT, RPT)])
        pltpu.sync_copy(src_hbm.at[wid], src_v)
        pltpu.sync_copy(dst_hbm.at[wid], dst_v)
        plsc.subcore_barrier()

        def gather(chunk, buf):
            pltpu.async_copy(p_hbm.at[src_v.at[chunk]], rows_v.at[buf],
                             gsems[buf])

        def gwait(chunk, buf):
            pltpu.make_async_copy(p_hbm.at[src_v.at[chunk]], rows_v.at[buf],
                                  gsems[buf]).wait()

        def scatter(chunk, buf):
            pltpu.async_copy(rows_v.at[buf], acc_sh.at[dst_v.at[chunk]],
                             ssems[buf], add=True)

        def swait(chunk, buf):
            pltpu.make_async_copy(rows_v.at[buf], acc_sh.at[dst_v.at[chunk]],
                                  ssems[buf]).wait()

        gather(0, 0)
        gather(1, 1)

        def body(g, carry):
            for b in range(NBUF):
                j = g * NBUF + b
                gwait(j, b)
                scatter(j, b)
                nxt = j + 2
                nb = (b + 2) % NBUF

                @pl.when(nxt < NCH)
                def _():
                    @pl.when(nxt >= NBUF)
                    def _():
                        swait(nxt - NBUF, nb)

                    gather(nxt, nb)

            return carry

        lax.fori_loop(0, NCH // NBUF, body, 0)
        for b in range(NBUF):
            swait(NCH - NBUF + b, b)
        plsc.subcore_barrier()
        pltpu.sync_copy(acc_sh.at[pl.ds(s * RPT, RPT)],
                        out_hbm.at[c, pl.ds(s * RPT, RPT)])

    return agg


def _make_deg():
    @functools.partial(
        pl.kernel,
        out_type=jax.ShapeDtypeStruct((NC, NP, LANES), _f32),
        mesh=_sc_mesh(),
        compiler_params=pltpu.CompilerParams(use_tc_tiling_on_sc=False),
        scratch_types=[
            pltpu.VMEM((NCH, CH), jnp.int32),
            pltpu.VMEM((CH, LANES), _f32),
            pltpu.VMEM_SHARED((NP, LANES), _f32),
            pltpu.SemaphoreType.DMA,
        ],
    )
    def deg(dst_hbm, ones_hbm, z_hbm, out_hbm, dst_v, ones_v, acc_sh, sem):
        c = lax.axis_index("c")
        s = lax.axis_index("s")
        wid = s * NC + c
        pltpu.sync_copy(z_hbm, acc_sh.at[pl.ds(s * RPT, RPT)])
        pltpu.sync_copy(ones_hbm, ones_v)
        pltpu.sync_copy(dst_hbm.at[wid], dst_v)
        plsc.subcore_barrier()

        DEPTH = 8

        def body(j, carry):
            pltpu.async_copy(ones_v, acc_sh.at[dst_v.at[j]], sem, add=True)

            @pl.when(j >= DEPTH)
            def _():
                pltpu.make_async_copy(ones_v, acc_sh.at[dst_v.at[j]],
                                      sem).wait()

            return carry

        lax.fori_loop(0, NCH, body, 0)

        def drain(j, carry):
            pltpu.make_async_copy(ones_v, acc_sh.at[dst_v.at[j]], sem).wait()
            return carry

        lax.fori_loop(0, DEPTH, drain, 0)
        plsc.subcore_barrier()
        pltpu.sync_copy(acc_sh.at[pl.ds(s * RPT, RPT)],
                        out_hbm.at[c, pl.ds(s * RPT, RPT)])

    return deg


def _srow(deg_blk):
    d = deg_blk[0, :, 0:1] + deg_blk[1, :, 0:1] + 1.0
    return lax.rsqrt(d)


def _k_first(x_ref, deg_ref, w_ref, b_ref, o_ref):
    s = _srow(deg_ref[...])
    m = jnp.dot(x_ref[...], w_ref[...], preferred_element_type=_f32)
    o_ref[...] = s * (m + b_ref[...])


def _k_mid(a_ref, p_ref, deg_ref, w_ref, b_ref, o_ref):
    s = _srow(deg_ref[...])
    aa = a_ref[...]
    agg = s * (aa[0] + aa[1] + p_ref[...])
    h = jnp.where(agg > 0, agg, jnp.exp(agg) - 1.0)
    m = jnp.dot(h, w_ref[...], preferred_element_type=_f32)
    o_ref[...] = s * (m + b_ref[...])


def _k_last(a_ref, p_ref, deg_ref, o_ref):
    s = _srow(deg_ref[...])
    aa = a_ref[...]
    z = s * (aa[0] + aa[1] + p_ref[...])
    col = lax.broadcasted_iota(jnp.int32, z.shape, 1)
    zm = jnp.where(col < 28, z, -1e30)
    mx = jnp.max(zm, axis=-1, keepdims=True)
    e = jnp.exp(zm - mx)
    o_ref[...] = e / jnp.sum(e, axis=-1, keepdims=True)


def _row_spec(d):
    return pl.BlockSpec((BLK, d), lambda i: (i, 0))


def _deg_spec():
    return pl.BlockSpec((NC, BLK, LANES), lambda i: (0, i, 0))


def _part_spec(d):
    return pl.BlockSpec((NC, BLK, d), lambda i: (0, i, 0))


def _full_spec(shape):
    return pl.BlockSpec(shape, lambda i: tuple(0 for _ in shape))


def _tc_first(x, deg, w, b):
    din, dout = w.shape
    return pl.pallas_call(
        _k_first,
        grid=(GRID,),
        in_specs=[_row_spec(din), _deg_spec(),
                  _full_spec((din, dout)), _full_spec((1, dout))],
        out_specs=_row_spec(dout),
        out_shape=jax.ShapeDtypeStruct((NP, dout), _f32),
    )(x, deg, w, b)


def _tc_mid(a, p, deg, w, b):
    din, dout = w.shape
    return pl.pallas_call(
        _k_mid,
        grid=(GRID,),
        in_specs=[_part_spec(din), _row_spec(din), _deg_spec(),
                  _full_spec((din, dout)), _full_spec((1, dout))],
        out_specs=_row_spec(dout),
        out_shape=jax.ShapeDtypeStruct((NP, dout), _f32),
    )(a, p, deg, w, b)


def _tc_last(a, p, deg):
    d = p.shape[1]
    return pl.pallas_call(
        _k_last,
        grid=(GRID,),
        in_specs=[_part_spec(d), _row_spec(d), _deg_spec()],
        out_specs=_row_spec(d),
        out_shape=jax.ShapeDtypeStruct((NP, d), _f32),
    )(a, p, deg)


def kernel(x, edge_index, W1, b1, W2, b2, W3, b3, W4, b4):
    ei = edge_index.astype(jnp.int32)
    pad = jnp.full((EPAD - E,), N, jnp.int32)
    src_r = jnp.concatenate([ei[0], pad]).reshape(NW, NCH, CH)
    dst_r = jnp.concatenate([ei[1], pad]).reshape(NW, NCH, CH)
    xp = jnp.pad(x, ((0, NP - N), (0, 0)))
    w4p = jnp.pad(W4, ((0, 0), (0, 4)))
    b4p = jnp.pad(b4, (0, 4))

    ones16 = jnp.ones((CH, LANES), _f32)
    z16 = jnp.zeros((RPT, LANES), _f32)
    z32 = jnp.zeros((RPT, 32), _f32)
    z64 = jnp.zeros((RPT, 64), _f32)

    deg = _make_deg()(dst_r, ones16, z16)
    agg32 = _make_agg(32)
    agg64 = _make_agg(64)

    p1 = _tc_first(xp, deg, W1, b1.reshape(1, -1))
    a1 = agg32(p1, src_r, dst_r, z32)
    p2 = _tc_mid(a1, p1, deg, W2, b2.reshape(1, -1))
    a2 = agg64(p2, src_r, dst_r, z64)
    p3 = _tc_mid(a2, p2, deg, W3, b3.reshape(1, -1))
    a3 = agg32(p3, src_r, dst_r, z32)
    p4 = _tc_mid(a3, p3, deg, w4p, b4p.reshape(1, -1))
    a4 = agg32(p4, src_r, dst_r, z32)
    out = _tc_last(a4, p4, deg)
    return out[:N, :28]


# speedup vs baseline: 28.6533x; 1.3693x over previous
import functools

import jax
import jax.numpy as jnp
from jax import lax
from jax.experimental import pallas as pl
from jax.experimental.pallas import tpu as pltpu
from jax.experimental.pallas import tpu_sc as plsc

N = 10000
NP = 10240
E = 320000
NC, NS, LANES = 2, 16, 16
NW = NC * NS
CH = 128
NCH = 80
EPW = NCH * CH
EPAD = NW * EPW
RPT = NP // NS
NBUF = 4
BLK = 1024
GRID = NP // BLK

_f32 = jnp.float32


def _sc_mesh():
    return plsc.VectorSubcoreMesh(
        core_axis_name="c", subcore_axis_name="s",
        num_cores=NC, num_subcores=NS)


def _make_agg(D, stage):
    scratch = [
        pltpu.VMEM((NCH, CH), jnp.int32),
        pltpu.VMEM((NCH, CH), jnp.int32),
        pltpu.VMEM((NBUF, CH, D), _f32),
        pltpu.VMEM_SHARED((NP, D), _f32),
        [pltpu.SemaphoreType.DMA] * NBUF,
        [pltpu.SemaphoreType.DMA] * NBUF,
    ]
    if stage:
        scratch.append(pltpu.VMEM_SHARED((NP, D), _f32))

    @functools.partial(
        pl.kernel,
        out_type=jax.ShapeDtypeStruct((NC, NP, D), _f32),
        mesh=_sc_mesh(),
        compiler_params=pltpu.CompilerParams(use_tc_tiling_on_sc=False),
        scratch_types=scratch,
    )
    def agg(p_hbm, src_hbm, dst_hbm, z_hbm, out_hbm,
            src_v, dst_v, rows_v, acc_sh, gsems, ssems, *maybe_psh):
        c = lax.axis_index("c")
        s = lax.axis_index("s")
        wid = s * NC + c
        if stage:
            p_src = maybe_psh[0]
            pltpu.sync_copy(p_hbm.at[pl.ds(s * RPT, RPT)],
                            p_src.at[pl.ds(s * RPT, RPT)])
        else:
            p_src = p_hbm
        pltpu.sync_copy(z_hbm, acc_sh.at[pl.ds(s * RPT, RPT)])
        pltpu.sync_copy(src_hbm.at[wid], src_v)
        pltpu.sync_copy(dst_hbm.at[wid], dst_v)
        plsc.subcore_barrier()

        def gather(chunk, buf):
            pltpu.async_copy(p_src.at[src_v.at[chunk]], rows_v.at[buf],
                             gsems[buf])

        def gwait(chunk, buf):
            pltpu.make_async_copy(p_src.at[src_v.at[chunk]], rows_v.at[buf],
                                  gsems[buf]).wait()

        def scatter(chunk, buf):
            pltpu.async_copy(rows_v.at[buf], acc_sh.at[dst_v.at[chunk]],
                             ssems[buf], add=True)

        def swait(chunk, buf):
            pltpu.make_async_copy(rows_v.at[buf], acc_sh.at[dst_v.at[chunk]],
                                  ssems[buf]).wait()

        gather(0, 0)
        gather(1, 1)

        def body(g, carry):
            for b in range(NBUF):
                j = g * NBUF + b
                gwait(j, b)
                scatter(j, b)
                nxt = j + 2
                nb = (b + 2) % NBUF

                @pl.when(nxt < NCH)
                def _():
                    @pl.when(nxt >= NBUF)
                    def _():
                        swait(nxt - NBUF, nb)

                    gather(nxt, nb)

            return carry

        lax.fori_loop(0, NCH // NBUF, body, 0)
        for b in range(NBUF):
            swait(NCH - NBUF + b, b)
        plsc.subcore_barrier()
        pltpu.sync_copy(acc_sh.at[pl.ds(s * RPT, RPT)],
                        out_hbm.at[c, pl.ds(s * RPT, RPT)])

    return agg


def _make_deg():
    @functools.partial(
        pl.kernel,
        out_type=jax.ShapeDtypeStruct((NC, NP, LANES), _f32),
        mesh=_sc_mesh(),
        compiler_params=pltpu.CompilerParams(use_tc_tiling_on_sc=False),
        scratch_types=[
            pltpu.VMEM((NCH, CH), jnp.int32),
            pltpu.VMEM((CH, LANES), _f32),
            pltpu.VMEM_SHARED((NP, LANES), _f32),
            pltpu.SemaphoreType.DMA,
        ],
    )
    def deg(dst_hbm, ones_hbm, z_hbm, out_hbm, dst_v, ones_v, acc_sh, sem):
        c = lax.axis_index("c")
        s = lax.axis_index("s")
        wid = s * NC + c
        pltpu.sync_copy(z_hbm, acc_sh.at[pl.ds(s * RPT, RPT)])
        pltpu.sync_copy(ones_hbm, ones_v)
        pltpu.sync_copy(dst_hbm.at[wid], dst_v)
        plsc.subcore_barrier()

        DEPTH = 8

        def body(j, carry):
            pltpu.async_copy(ones_v, acc_sh.at[dst_v.at[j]], sem, add=True)

            @pl.when(j >= DEPTH)
            def _():
                pltpu.make_async_copy(ones_v, acc_sh.at[dst_v.at[j]],
                                      sem).wait()

            return carry

        lax.fori_loop(0, NCH, body, 0)

        def drain(j, carry):
            pltpu.make_async_copy(ones_v, acc_sh.at[dst_v.at[j]], sem).wait()
            return carry

        lax.fori_loop(0, DEPTH, drain, 0)
        plsc.subcore_barrier()
        pltpu.sync_copy(acc_sh.at[pl.ds(s * RPT, RPT)],
                        out_hbm.at[c, pl.ds(s * RPT, RPT)])

    return deg


def _srow(deg_blk):
    d = deg_blk[0, :, 0:1] + deg_blk[1, :, 0:1] + 1.0
    return lax.rsqrt(d)


def _k_first(x_ref, deg_ref, w_ref, b_ref, o_ref):
    s = _srow(deg_ref[...])
    m = jnp.dot(x_ref[...], w_ref[...], preferred_element_type=_f32)
    o_ref[...] = s * (m + b_ref[...])


def _k_mid(a_ref, p_ref, deg_ref, w_ref, b_ref, o_ref):
    s = _srow(deg_ref[...])
    aa = a_ref[...]
    agg = s * (aa[0] + aa[1] + p_ref[...])
    h = jnp.where(agg > 0, agg, jnp.exp(agg) - 1.0)
    m = jnp.dot(h, w_ref[...], preferred_element_type=_f32)
    o_ref[...] = s * (m + b_ref[...])


def _k_last(a_ref, p_ref, deg_ref, o_ref):
    s = _srow(deg_ref[...])
    aa = a_ref[...]
    z = s * (aa[0] + aa[1] + p_ref[...])
    col = lax.broadcasted_iota(jnp.int32, z.shape, 1)
    zm = jnp.where(col < 28, z, -1e30)
    mx = jnp.max(zm, axis=-1, keepdims=True)
    e = jnp.exp(zm - mx)
    o_ref[...] = e / jnp.sum(e, axis=-1, keepdims=True)


def _row_spec(d):
    return pl.BlockSpec((BLK, d), lambda i: (i, 0))


def _deg_spec():
    return pl.BlockSpec((NC, BLK, LANES), lambda i: (0, i, 0))


def _part_spec(d):
    return pl.BlockSpec((NC, BLK, d), lambda i: (0, i, 0))


def _full_spec(shape):
    return pl.BlockSpec(shape, lambda i: tuple(0 for _ in shape))


def _tc_first(x, deg, w, b):
    din, dout = w.shape
    return pl.pallas_call(
        _k_first,
        grid=(GRID,),
        in_specs=[_row_spec(din), _deg_spec(),
                  _full_spec((din, dout)), _full_spec((1, dout))],
        out_specs=_row_spec(dout),
        out_shape=jax.ShapeDtypeStruct((NP, dout), _f32),
    )(x, deg, w, b)


def _tc_mid(a, p, deg, w, b):
    din, dout = w.shape
    return pl.pallas_call(
        _k_mid,
        grid=(GRID,),
        in_specs=[_part_spec(din), _row_spec(din), _deg_spec(),
                  _full_spec((din, dout)), _full_spec((1, dout))],
        out_specs=_row_spec(dout),
        out_shape=jax.ShapeDtypeStruct((NP, dout), _f32),
    )(a, p, deg, w, b)


def _tc_last(a, p, deg):
    d = p.shape[1]
    return pl.pallas_call(
        _k_last,
        grid=(GRID,),
        in_specs=[_part_spec(d), _row_spec(d), _deg_spec()],
        out_specs=_row_spec(d),
        out_shape=jax.ShapeDtypeStruct((NP, d), _f32),
    )(a, p, deg)


def kernel(x, edge_index, W1, b1, W2, b2, W3, b3, W4, b4):
    ei = edge_index.astype(jnp.int32)
    pad = jnp.full((EPAD - E,), N, jnp.int32)
    src_r = jnp.concatenate([ei[0], pad]).reshape(NW, NCH, CH)
    dst_r = jnp.concatenate([ei[1], pad]).reshape(NW, NCH, CH)
    xp = jnp.pad(x, ((0, NP - N), (0, 0)))
    w4p = jnp.pad(W4, ((0, 0), (0, 4)))
    b4p = jnp.pad(b4, (0, 4))

    ones16 = jnp.ones((CH, LANES), _f32)
    z16 = jnp.zeros((RPT, LANES), _f32)
    z32 = jnp.zeros((RPT, 32), _f32)
    z64 = jnp.zeros((RPT, 64), _f32)

    deg = _make_deg()(dst_r, ones16, z16)
    agg32 = _make_agg(32, stage=True)
    agg64 = _make_agg(64, stage=False)

    p1 = _tc_first(xp, deg, W1, b1.reshape(1, -1))
    a1 = agg32(p1, src_r, dst_r, z32)
    p2 = _tc_mid(a1, p1, deg, W2, b2.reshape(1, -1))
    a2 = agg64(p2, src_r, dst_r, z64)
    p3 = _tc_mid(a2, p2, deg, W3, b3.reshape(1, -1))
    a3 = agg32(p3, src_r, dst_r, z32)
    p4 = _tc_mid(a3, p3, deg, w4p, b4p.reshape(1, -1))
    a4 = agg32(p4, src_r, dst_r, z32)
    out = _tc_last(a4, p4, deg)
    return out[:N, :28]


# speedup vs baseline: 36.1339x; 1.2611x over previous
import functools

import jax
import jax.numpy as jnp
from jax import lax
from jax.experimental import pallas as pl
from jax.experimental.pallas import tpu as pltpu
from jax.experimental.pallas import tpu_sc as plsc

N = 10000
NP = 10240
E = 320000
NC, NS, LANES = 2, 16, 16
NW = NC * NS
CH = 128
NCH = 80
EPW = NCH * CH
EPAD = NW * EPW
RPT = NP // NS
NBUF = 4
BLK = 1024
GRID = NP // BLK

_f32 = jnp.float32


def _sc_mesh():
    return plsc.VectorSubcoreMesh(
        core_axis_name="c", subcore_axis_name="s",
        num_cores=NC, num_subcores=NS)


def _make_agg(D, stage):
    scratch = [
        pltpu.VMEM((NCH, CH), jnp.int32),
        pltpu.VMEM((NCH, CH), jnp.int32),
        pltpu.VMEM((NBUF, CH, D), _f32),
        pltpu.VMEM_SHARED((NP, D), _f32),
        [pltpu.SemaphoreType.DMA] * NBUF,
        [pltpu.SemaphoreType.DMA] * NBUF,
    ]
    if stage:
        scratch.append(pltpu.VMEM_SHARED((NP, D), _f32))

    @functools.partial(
        pl.kernel,
        out_type=jax.ShapeDtypeStruct((NC, NP, D), _f32),
        mesh=_sc_mesh(),
        compiler_params=pltpu.CompilerParams(use_tc_tiling_on_sc=False),
        scratch_types=scratch,
    )
    def agg(p_hbm, src_hbm, dst_hbm, z_hbm, out_hbm,
            src_v, dst_v, rows_v, acc_sh, gsems, ssems, *maybe_psh):
        c = lax.axis_index("c")
        s = lax.axis_index("s")
        wid = s * NC + c
        if stage:
            p_src = maybe_psh[0]
            pltpu.sync_copy(p_hbm.at[pl.ds(s * RPT, RPT)],
                            p_src.at[pl.ds(s * RPT, RPT)])
        else:
            p_src = p_hbm
        pltpu.sync_copy(z_hbm, acc_sh.at[pl.ds(s * RPT, RPT)])
        pltpu.sync_copy(src_hbm.at[wid], src_v)
        pltpu.sync_copy(dst_hbm.at[wid], dst_v)
        plsc.subcore_barrier()

        def gather(chunk, buf):
            pltpu.async_copy(p_src.at[src_v.at[chunk]], rows_v.at[buf],
                             gsems[buf])

        def gwait(chunk, buf):
            pltpu.make_async_copy(p_src.at[src_v.at[chunk]], rows_v.at[buf],
                                  gsems[buf]).wait()

        def scatter(chunk, buf):
            pltpu.async_copy(rows_v.at[buf], acc_sh.at[dst_v.at[chunk]],
                             ssems[buf], add=True)

        def swait(chunk, buf):
            pltpu.make_async_copy(rows_v.at[buf], acc_sh.at[dst_v.at[chunk]],
                                  ssems[buf]).wait()

        gather(0, 0)
        gather(1, 1)

        def body(g, carry):
            for b in range(NBUF):
                j = g * NBUF + b
                gwait(j, b)
                scatter(j, b)
                nxt = j + 2
                nb = (b + 2) % NBUF

                @pl.when(nxt < NCH)
                def _():
                    @pl.when(nxt >= NBUF)
                    def _():
                        swait(nxt - NBUF, nb)

                    gather(nxt, nb)

            return carry

        lax.fori_loop(0, NCH // NBUF, body, 0)
        for b in range(NBUF):
            swait(NCH - NBUF + b, b)
        plsc.subcore_barrier()
        pltpu.sync_copy(acc_sh.at[pl.ds(s * RPT, RPT)],
                        out_hbm.at[c, pl.ds(s * RPT, RPT)])

    return agg


def _make_deg():
    @functools.partial(
        pl.kernel,
        out_type=jax.ShapeDtypeStruct((NC, NP, LANES), _f32),
        mesh=_sc_mesh(),
        compiler_params=pltpu.CompilerParams(use_tc_tiling_on_sc=False),
        scratch_types=[
            pltpu.VMEM((NCH, CH), jnp.int32),
            pltpu.VMEM((CH, LANES), _f32),
            pltpu.VMEM_SHARED((NP, LANES), _f32),
            pltpu.SemaphoreType.DMA,
        ],
    )
    def deg(dst_hbm, ones_hbm, z_hbm, out_hbm, dst_v, ones_v, acc_sh, sem):
        c = lax.axis_index("c")
        s = lax.axis_index("s")
        wid = s * NC + c
        pltpu.sync_copy(z_hbm, acc_sh.at[pl.ds(s * RPT, RPT)])
        pltpu.sync_copy(ones_hbm, ones_v)
        pltpu.sync_copy(dst_hbm.at[wid], dst_v)
        plsc.subcore_barrier()

        DEPTH = 8

        def body(j, carry):
            pltpu.async_copy(ones_v, acc_sh.at[dst_v.at[j]], sem, add=True)

            @pl.when(j >= DEPTH)
            def _():
                pltpu.make_async_copy(ones_v, acc_sh.at[dst_v.at[j]],
                                      sem).wait()

            return carry

        lax.fori_loop(0, NCH, body, 0)

        def drain(j, carry):
            pltpu.make_async_copy(ones_v, acc_sh.at[dst_v.at[j]], sem).wait()
            return carry

        lax.fori_loop(0, DEPTH, drain, 0)
        plsc.subcore_barrier()
        pltpu.sync_copy(acc_sh.at[pl.ds(s * RPT, RPT)],
                        out_hbm.at[c, pl.ds(s * RPT, RPT)])

    return deg


def _srow(deg_blk):
    d = deg_blk[0, :, 0:1] + deg_blk[1, :, 0:1] + 1.0
    return lax.rsqrt(d)


def _k_first(x_ref, deg_ref, w_ref, b_ref, o_ref):
    s = _srow(deg_ref[...])
    m = jnp.dot(x_ref[...], w_ref[...], preferred_element_type=_f32)
    o_ref[...] = s * (m + b_ref[...])


def _k_mid(a_ref, p_ref, deg_ref, w_ref, b_ref, o_ref):
    s = _srow(deg_ref[...])
    aa = a_ref[...]
    agg = s * (aa[0] + aa[1] + p_ref[...])
    h = jnp.where(agg > 0, agg, jnp.exp(agg) - 1.0)
    m = jnp.dot(h, w_ref[...], preferred_element_type=_f32)
    o_ref[...] = s * (m + b_ref[...])


def _k_mid_split(a_ref, p_ref, deg_ref, w_ref, b_ref, o1_ref, o2_ref):
    s = _srow(deg_ref[...])
    aa = a_ref[...]
    agg = s * (aa[0] + aa[1] + p_ref[...])
    h = jnp.where(agg > 0, agg, jnp.exp(agg) - 1.0)
    m = jnp.dot(h, w_ref[...], preferred_element_type=_f32)
    res = s * (m + b_ref[...])
    o1_ref[...] = res[:, :32]
    o2_ref[...] = res[:, 32:]


def _k_mid_cat(aa_ref, ab_ref, pa_ref, pb_ref, deg_ref, w_ref, b_ref, o_ref):
    s = _srow(deg_ref[...])
    va = aa_ref[...]
    vb = ab_ref[...]
    ca = va[0] + va[1] + pa_ref[...]
    cb = vb[0] + vb[1] + pb_ref[...]
    agg = s * jnp.concatenate([ca, cb], axis=1)
    h = jnp.where(agg > 0, agg, jnp.exp(agg) - 1.0)
    m = jnp.dot(h, w_ref[...], preferred_element_type=_f32)
    o_ref[...] = s * (m + b_ref[...])


def _k_last(a_ref, p_ref, deg_ref, o_ref):
    s = _srow(deg_ref[...])
    aa = a_ref[...]
    z = s * (aa[0] + aa[1] + p_ref[...])
    col = lax.broadcasted_iota(jnp.int32, z.shape, 1)
    zm = jnp.where(col < 28, z, -1e30)
    mx = jnp.max(zm, axis=-1, keepdims=True)
    e = jnp.exp(zm - mx)
    o_ref[...] = e / jnp.sum(e, axis=-1, keepdims=True)


def _row_spec(d):
    return pl.BlockSpec((BLK, d), lambda i: (i, 0))


def _deg_spec():
    return pl.BlockSpec((NC, BLK, LANES), lambda i: (0, i, 0))


def _part_spec(d):
    return pl.BlockSpec((NC, BLK, d), lambda i: (0, i, 0))


def _full_spec(shape):
    return pl.BlockSpec(shape, lambda i: tuple(0 for _ in shape))


def _tc_first(x, deg, w, b):
    din, dout = w.shape
    return pl.pallas_call(
        _k_first,
        grid=(GRID,),
        in_specs=[_row_spec(din), _deg_spec(),
                  _full_spec((din, dout)), _full_spec((1, dout))],
        out_specs=_row_spec(dout),
        out_shape=jax.ShapeDtypeStruct((NP, dout), _f32),
    )(x, deg, w, b)


def _tc_mid(a, p, deg, w, b):
    din, dout = w.shape
    return pl.pallas_call(
        _k_mid,
        grid=(GRID,),
        in_specs=[_part_spec(din), _row_spec(din), _deg_spec(),
                  _full_spec((din, dout)), _full_spec((1, dout))],
        out_specs=_row_spec(dout),
        out_shape=jax.ShapeDtypeStruct((NP, dout), _f32),
    )(a, p, deg, w, b)


def _tc_mid_split(a, p, deg, w, b):
    din, dout = w.shape
    half = dout // 2
    return pl.pallas_call(
        _k_mid_split,
        grid=(GRID,),
        in_specs=[_part_spec(din), _row_spec(din), _deg_spec(),
                  _full_spec((din, dout)), _full_spec((1, dout))],
        out_specs=[_row_spec(half), _row_spec(half)],
        out_shape=[jax.ShapeDtypeStruct((NP, half), _f32),
                   jax.ShapeDtypeStruct((NP, half), _f32)],
    )(a, p, deg, w, b)


def _tc_mid_cat(aa, ab, pa, pb, deg, w, b):
    din, dout = w.shape
    half = din // 2
    return pl.pallas_call(
        _k_mid_cat,
        grid=(GRID,),
        in_specs=[_part_spec(half), _part_spec(half),
                  _row_spec(half), _row_spec(half), _deg_spec(),
                  _full_spec((din, dout)), _full_spec((1, dout))],
        out_specs=_row_spec(dout),
        out_shape=jax.ShapeDtypeStruct((NP, dout), _f32),
    )(aa, ab, pa, pb, deg, w, b)


def _tc_last(a, p, deg):
    d = p.shape[1]
    return pl.pallas_call(
        _k_last,
        grid=(GRID,),
        in_specs=[_part_spec(d), _row_spec(d), _deg_spec()],
        out_specs=_row_spec(d),
        out_shape=jax.ShapeDtypeStruct((NP, d), _f32),
    )(a, p, deg)


def kernel(x, edge_index, W1, b1, W2, b2, W3, b3, W4, b4):
    ei = edge_index.astype(jnp.int32)
    pad = jnp.full((EPAD - E,), N, jnp.int32)
    src_r = jnp.concatenate([ei[0], pad]).reshape(NW, NCH, CH)
    dst_r = jnp.concatenate([ei[1], pad]).reshape(NW, NCH, CH)
    xp = jnp.pad(x, ((0, NP - N), (0, 0)))
    w4p = jnp.pad(W4, ((0, 0), (0, 4)))
    b4p = jnp.pad(b4, (0, 4))

    ones16 = jnp.ones((CH, LANES), _f32)
    z16 = jnp.zeros((RPT, LANES), _f32)
    z32 = jnp.zeros((RPT, 32), _f32)
    z64 = jnp.zeros((RPT, 64), _f32)

    deg = _make_deg()(dst_r, ones16, z16)
    agg32 = _make_agg(32, stage=True)

    p1 = _tc_first(xp, deg, W1, b1.reshape(1, -1))
    a1 = agg32(p1, src_r, dst_r, z32)
    p2a, p2b = _tc_mid_split(a1, p1, deg, W2, b2.reshape(1, -1))
    a2a = agg32(p2a, src_r, dst_r, z32)
    a2b = agg32(p2b, src_r, dst_r, z32)
    p3 = _tc_mid_cat(a2a, a2b, p2a, p2b, deg, W3, b3.reshape(1, -1))
    a3 = agg32(p3, src_r, dst_r, z32)
    p4 = _tc_mid(a3, p3, deg, w4p, b4p.reshape(1, -1))
    a4 = agg32(p4, src_r, dst_r, z32)
    out = _tc_last(a4, p4, deg)
    return out[:N, :28]


# speedup vs baseline: 36.1622x; 1.0008x over previous
import functools

import jax
import jax.numpy as jnp
from jax import lax
from jax.experimental import pallas as pl
from jax.experimental.pallas import tpu as pltpu
from jax.experimental.pallas import tpu_sc as plsc

N = 10000
NP = 10240
E = 320000
NC, NS, LANES = 2, 16, 16
NW = NC * NS
CH = 128
NCH = 80
EPAD = NW * NCH * CH
RPT = NP // NS
NBUF = 4
BLK = 1024
GRID = NP // BLK

_f32 = jnp.float32


def _sc_mesh():
    return plsc.VectorSubcoreMesh(
        core_axis_name="c", subcore_axis_name="s",
        num_cores=NC, num_subcores=NS)


def _edge_pass(p_src, src_v, dst_v, rows_v, acc_sh, gsems, ssems):
    def gather(chunk, buf):
        pltpu.async_copy(p_src.at[src_v.at[chunk]], rows_v.at[buf],
                         gsems[buf])

    def gwait(chunk, buf):
        pltpu.make_async_copy(p_src.at[src_v.at[chunk]], rows_v.at[buf],
                              gsems[buf]).wait()

    def scatter(chunk, buf):
        pltpu.async_copy(rows_v.at[buf], acc_sh.at[dst_v.at[chunk]],
                         ssems[buf], add=True)

    def swait(chunk, buf):
        pltpu.make_async_copy(rows_v.at[buf], acc_sh.at[dst_v.at[chunk]],
                              ssems[buf]).wait()

    gather(0, 0)
    gather(1, 1)

    def body(g, carry):
        for b in range(NBUF):
            j = g * NBUF + b
            gwait(j, b)
            scatter(j, b)
            nxt = j + 2
            nb = (b + 2) % NBUF

            @pl.when(nxt < NCH)
            def _():
                @pl.when(nxt >= NBUF)
                def _():
                    swait(nxt - NBUF, nb)

                gather(nxt, nb)

        return carry

    lax.fori_loop(0, NCH // NBUF, body, 0)
    for b in range(NBUF):
        swait(NCH - NBUF + b, b)


def _make_agg(D):
    @functools.partial(
        pl.kernel,
        out_type=jax.ShapeDtypeStruct((NC, NP, D), _f32),
        mesh=_sc_mesh(),
        compiler_params=pltpu.CompilerParams(use_tc_tiling_on_sc=False),
        scratch_types=[
            pltpu.VMEM((NCH, CH), jnp.int32),
            pltpu.VMEM((NCH, CH), jnp.int32),
            pltpu.VMEM((NBUF, CH, D), _f32),
            pltpu.VMEM_SHARED((NP, D), _f32),
            pltpu.VMEM_SHARED((NP, D), _f32),
            [pltpu.SemaphoreType.DMA] * NBUF,
            [pltpu.SemaphoreType.DMA] * NBUF,
        ],
    )
    def agg(p_hbm, src_hbm, dst_hbm, z_hbm, out_hbm,
            src_v, dst_v, rows_v, acc_sh, p_sh, gsems, ssems):
        c = lax.axis_index("c")
        s = lax.axis_index("s")
        wid = s * NC + c
        rows = pl.ds(s * RPT, RPT)
        pltpu.sync_copy(p_hbm.at[rows], p_sh.at[rows])
        pltpu.sync_copy(z_hbm, acc_sh.at[rows])
        pltpu.sync_copy(src_hbm.at[wid], src_v)
        pltpu.sync_copy(dst_hbm.at[wid], dst_v)
        plsc.subcore_barrier()
        _edge_pass(p_sh, src_v, dst_v, rows_v, acc_sh, gsems, ssems)
        plsc.subcore_barrier()
        pltpu.sync_copy(acc_sh.at[rows], out_hbm.at[c, rows])

    return agg


def _make_agg_dual(D):
    @functools.partial(
        pl.kernel,
        out_type=[jax.ShapeDtypeStruct((NC, NP, D), _f32),
                  jax.ShapeDtypeStruct((NC, NP, D), _f32)],
        mesh=_sc_mesh(),
        compiler_params=pltpu.CompilerParams(use_tc_tiling_on_sc=False),
        scratch_types=[
            pltpu.VMEM((NCH, CH), jnp.int32),
            pltpu.VMEM((NCH, CH), jnp.int32),
            pltpu.VMEM((NBUF, CH, D), _f32),
            pltpu.VMEM_SHARED((NP, D), _f32),
            pltpu.VMEM_SHARED((NP, D), _f32),
            [pltpu.SemaphoreType.DMA] * NBUF,
            [pltpu.SemaphoreType.DMA] * NBUF,
        ],
    )
    def agg2(pa_hbm, pb_hbm, src_hbm, dst_hbm, z_hbm, outa_hbm, outb_hbm,
             src_v, dst_v, rows_v, acc_sh, p_sh, gsems, ssems):
        c = lax.axis_index("c")
        s = lax.axis_index("s")
        wid = s * NC + c
        rows = pl.ds(s * RPT, RPT)
        pltpu.sync_copy(src_hbm.at[wid], src_v)
        pltpu.sync_copy(dst_hbm.at[wid], dst_v)
        for p_hbm, out_hbm in ((pa_hbm, outa_hbm), (pb_hbm, outb_hbm)):
            pltpu.sync_copy(p_hbm.at[rows], p_sh.at[rows])
            pltpu.sync_copy(z_hbm, acc_sh.at[rows])
            plsc.subcore_barrier()
            _edge_pass(p_sh, src_v, dst_v, rows_v, acc_sh, gsems, ssems)
            plsc.subcore_barrier()
            pltpu.sync_copy(acc_sh.at[rows], out_hbm.at[c, rows])
            plsc.subcore_barrier()

    return agg2


def _make_deg():
    @functools.partial(
        pl.kernel,
        out_type=jax.ShapeDtypeStruct((NC, NP, LANES), _f32),
        mesh=_sc_mesh(),
        compiler_params=pltpu.CompilerParams(use_tc_tiling_on_sc=False),
        scratch_types=[
            pltpu.VMEM((NCH, CH), jnp.int32),
            pltpu.VMEM((CH, LANES), _f32),
            pltpu.VMEM_SHARED((NP, LANES), _f32),
            pltpu.SemaphoreType.DMA,
        ],
    )
    def deg(dst_hbm, ones_hbm, z_hbm, out_hbm, dst_v, ones_v, acc_sh, sem):
        c = lax.axis_index("c")
        s = lax.axis_index("s")
        wid = s * NC + c
        pltpu.sync_copy(z_hbm, acc_sh.at[pl.ds(s * RPT, RPT)])
        pltpu.sync_copy(ones_hbm, ones_v)
        pltpu.sync_copy(dst_hbm.at[wid], dst_v)
        plsc.subcore_barrier()

        DEPTH = 8

        def body(j, carry):
            pltpu.async_copy(ones_v, acc_sh.at[dst_v.at[j]], sem, add=True)

            @pl.when(j >= DEPTH)
            def _():
                pltpu.make_async_copy(ones_v, acc_sh.at[dst_v.at[j]],
                                      sem).wait()

            return carry

        lax.fori_loop(0, NCH, body, 0)

        def drain(j, carry):
            pltpu.make_async_copy(ones_v, acc_sh.at[dst_v.at[j]], sem).wait()
            return carry

        lax.fori_loop(0, DEPTH, drain, 0)
        plsc.subcore_barrier()
        pltpu.sync_copy(acc_sh.at[pl.ds(s * RPT, RPT)],
                        out_hbm.at[c, pl.ds(s * RPT, RPT)])

    return deg


def _srow(deg_blk):
    d = deg_blk[0, :, 0:1] + deg_blk[1, :, 0:1] + 1.0
    return lax.rsqrt(d)


def _k_first(x_ref, deg_ref, w_ref, b_ref, o_ref):
    s = _srow(deg_ref[...])
    m = jnp.dot(x_ref[...], w_ref[...], preferred_element_type=_f32)
    o_ref[...] = s * (m + b_ref[...])


def _k_mid(a_ref, p_ref, deg_ref, w_ref, b_ref, o_ref):
    s = _srow(deg_ref[...])
    aa = a_ref[...]
    agg = s * (aa[0] + aa[1] + p_ref[...])
    h = jnp.where(agg > 0, agg, jnp.exp(agg) - 1.0)
    m = jnp.dot(h, w_ref[...], preferred_element_type=_f32)
    o_ref[...] = s * (m + b_ref[...])


def _k_mid_split(a_ref, p_ref, deg_ref, w_ref, b_ref, o1_ref, o2_ref):
    s = _srow(deg_ref[...])
    aa = a_ref[...]
    agg = s * (aa[0] + aa[1] + p_ref[...])
    h = jnp.where(agg > 0, agg, jnp.exp(agg) - 1.0)
    m = jnp.dot(h, w_ref[...], preferred_element_type=_f32)
    res = s * (m + b_ref[...])
    o1_ref[...] = res[:, :32]
    o2_ref[...] = res[:, 32:]


def _k_mid_cat(aa_ref, ab_ref, pa_ref, pb_ref, deg_ref, w_ref, b_ref, o_ref):
    s = _srow(deg_ref[...])
    va = aa_ref[...]
    vb = ab_ref[...]
    ca = va[0] + va[1] + pa_ref[...]
    cb = vb[0] + vb[1] + pb_ref[...]
    agg = s * jnp.concatenate([ca, cb], axis=1)
    h = jnp.where(agg > 0, agg, jnp.exp(agg) - 1.0)
    m = jnp.dot(h, w_ref[...], preferred_element_type=_f32)
    o_ref[...] = s * (m + b_ref[...])


def _k_last(a_ref, p_ref, deg_ref, o_ref):
    s = _srow(deg_ref[...])
    aa = a_ref[...]
    z = s * (aa[0] + aa[1] + p_ref[...])
    col = lax.broadcasted_iota(jnp.int32, z.shape, 1)
    zm = jnp.where(col < 28, z, -1e30)
    mx = jnp.max(zm, axis=-1, keepdims=True)
    e = jnp.exp(zm - mx)
    o_ref[...] = e / jnp.sum(e, axis=-1, keepdims=True)


def _row_spec(d):
    return pl.BlockSpec((BLK, d), lambda i: (i, 0))


def _deg_spec():
    return pl.BlockSpec((NC, BLK, LANES), lambda i: (0, i, 0))


def _part_spec(d):
    return pl.BlockSpec((NC, BLK, d), lambda i: (0, i, 0))


def _full_spec(shape):
    return pl.BlockSpec(shape, lambda i: tuple(0 for _ in shape))


def _tc_first(x, deg, w, b):
    din, dout = w.shape
    return pl.pallas_call(
        _k_first,
        grid=(GRID,),
        in_specs=[_row_spec(din), _deg_spec(),
                  _full_spec((din, dout)), _full_spec((1, dout))],
        out_specs=_row_spec(dout),
        out_shape=jax.ShapeDtypeStruct((NP, dout), _f32),
    )(x, deg, w, b)


def _tc_mid(a, p, deg, w, b):
    din, dout = w.shape
    return pl.pallas_call(
        _k_mid,
        grid=(GRID,),
        in_specs=[_part_spec(din), _row_spec(din), _deg_spec(),
                  _full_spec((din, dout)), _full_spec((1, dout))],
        out_specs=_row_spec(dout),
        out_shape=jax.ShapeDtypeStruct((NP, dout), _f32),
    )(a, p, deg, w, b)


def _tc_mid_split(a, p, deg, w, b):
    din, dout = w.shape
    half = dout // 2
    return pl.pallas_call(
        _k_mid_split,
        grid=(GRID,),
        in_specs=[_part_spec(din), _row_spec(din), _deg_spec(),
                  _full_spec((din, dout)), _full_spec((1, dout))],
        out_specs=[_row_spec(half), _row_spec(half)],
        out_shape=[jax.ShapeDtypeStruct((NP, half), _f32),
                   jax.ShapeDtypeStruct((NP, half), _f32)],
    )(a, p, deg, w, b)


def _tc_mid_cat(aa, ab, pa, pb, deg, w, b):
    din, dout = w.shape
    half = din // 2
    return pl.pallas_call(
        _k_mid_cat,
        grid=(GRID,),
        in_specs=[_part_spec(half), _part_spec(half),
                  _row_spec(half), _row_spec(half), _deg_spec(),
                  _full_spec((din, dout)), _full_spec((1, dout))],
        out_specs=_row_spec(dout),
        out_shape=jax.ShapeDtypeStruct((NP, dout), _f32),
    )(aa, ab, pa, pb, deg, w, b)


def _tc_last(a, p, deg):
    d = p.shape[1]
    return pl.pallas_call(
        _k_last,
        grid=(GRID,),
        in_specs=[_part_spec(d), _row_spec(d), _deg_spec()],
        out_specs=_row_spec(d),
        out_shape=jax.ShapeDtypeStruct((NP, d), _f32),
    )(a, p, deg)


def kernel(x, edge_index, W1, b1, W2, b2, W3, b3, W4, b4):
    ei = edge_index.astype(jnp.int32)
    pad = jnp.full((EPAD - E,), N, jnp.int32)
    src_r = jnp.concatenate([ei[0], pad]).reshape(NW, NCH, CH)
    dst_r = jnp.concatenate([ei[1], pad]).reshape(NW, NCH, CH)
    xp = jnp.pad(x, ((0, NP - N), (0, 0)))
    w4p = jnp.pad(W4, ((0, 0), (0, 4)))
    b4p = jnp.pad(b4, (0, 4))

    ones16 = jnp.ones((CH, LANES), _f32)
    z16 = jnp.zeros((RPT, LANES), _f32)
    z32 = jnp.zeros((RPT, 32), _f32)

    deg = _make_deg()(dst_r, ones16, z16)
    agg32 = _make_agg(32)
    agg32x2 = _make_agg_dual(32)

    p1 = _tc_first(xp, deg, W1, b1.reshape(1, -1))
    a1 = agg32(p1, src_r, dst_r, z32)
    p2a, p2b = _tc_mid_split(a1, p1, deg, W2, b2.reshape(1, -1))
    a2a, a2b = agg32x2(p2a, p2b, src_r, dst_r, z32)
    p3 = _tc_mid_cat(a2a, a2b, p2a, p2b, deg, W3, b3.reshape(1, -1))
    a3 = agg32(p3, src_r, dst_r, z32)
    p4 = _tc_mid(a3, p3, deg, w4p, b4p.reshape(1, -1))
    a4 = agg32(p4, src_r, dst_r, z32)
    out = _tc_last(a4, p4, deg)
    return out[:N, :28]


# speedup vs baseline: 46.2101x; 1.2779x over previous
import functools

import jax
import jax.numpy as jnp
from jax import lax
from jax.experimental import pallas as pl
from jax.experimental.pallas import tpu as pltpu
from jax.experimental.pallas import tpu_sc as plsc

N = 10000
NP = 10240
E = 320000
NC, NS, LANES = 2, 16, 16
NW = NC * NS
CH = 128
NCH = 80
EPAD = NW * NCH * CH
RPT = NP // NS
NBUF = 4
BLK = 1024
GRID = NP // BLK
BLK4 = BLK // 4
NP4 = NP // 4

_f32 = jnp.float32


def _sc_mesh():
    return plsc.VectorSubcoreMesh(
        core_axis_name="c", subcore_axis_name="s",
        num_cores=NC, num_subcores=NS)


def _edge_pass(p_src, src_v, dst_v, rows_v, acc_sh, gsems, ssems):
    def gather(chunk, buf):
        pltpu.async_copy(p_src.at[src_v.at[chunk]], rows_v.at[buf],
                         gsems[buf])

    def gwait(chunk, buf):
        pltpu.make_async_copy(p_src.at[src_v.at[chunk]], rows_v.at[buf],
                              gsems[buf]).wait()

    def scatter(chunk, buf):
        pltpu.async_copy(rows_v.at[buf], acc_sh.at[dst_v.at[chunk]],
                         ssems[buf], add=True)

    def swait(chunk, buf):
        pltpu.make_async_copy(rows_v.at[buf], acc_sh.at[dst_v.at[chunk]],
                              ssems[buf]).wait()

    gather(0, 0)
    gather(1, 1)

    def body(g, carry):
        for b in range(NBUF):
            j = g * NBUF + b
            gwait(j, b)
            scatter(j, b)
            nxt = j + 2
            nb = (b + 2) % NBUF

            @pl.when(nxt < NCH)
            def _():
                @pl.when(nxt >= NBUF)
                def _():
                    swait(nxt - NBUF, nb)

                gather(nxt, nb)

        return carry

    lax.fori_loop(0, NCH // NBUF, body, 0)
    for b in range(NBUF):
        swait(NCH - NBUF + b, b)


def _make_agg(D):
    @functools.partial(
        pl.kernel,
        out_type=jax.ShapeDtypeStruct((NC, NP, D), _f32),
        mesh=_sc_mesh(),
        compiler_params=pltpu.CompilerParams(use_tc_tiling_on_sc=False),
        scratch_types=[
            pltpu.VMEM((NCH, CH), jnp.int32),
            pltpu.VMEM((NCH, CH), jnp.int32),
            pltpu.VMEM((NBUF, CH, D), _f32),
            pltpu.VMEM_SHARED((NP, D), _f32),
            pltpu.VMEM_SHARED((NP, D), _f32),
            [pltpu.SemaphoreType.DMA] * NBUF,
            [pltpu.SemaphoreType.DMA] * NBUF,
        ],
    )
    def agg(p_hbm, src_hbm, dst_hbm, z_hbm, out_hbm,
            src_v, dst_v, rows_v, acc_sh, p_sh, gsems, ssems):
        c = lax.axis_index("c")
        s = lax.axis_index("s")
        wid = s * NC + c
        rows = pl.ds(s * RPT, RPT)
        pltpu.sync_copy(p_hbm.at[rows], p_sh.at[rows])
        pltpu.sync_copy(z_hbm, acc_sh.at[rows])
        pltpu.sync_copy(src_hbm.at[wid], src_v)
        pltpu.sync_copy(dst_hbm.at[wid], dst_v)
        plsc.subcore_barrier()
        _edge_pass(p_sh, src_v, dst_v, rows_v, acc_sh, gsems, ssems)
        plsc.subcore_barrier()
        pltpu.sync_copy(acc_sh.at[rows], out_hbm.at[c, rows])

    return agg


def _make_agg_dual(D):
    @functools.partial(
        pl.kernel,
        out_type=[jax.ShapeDtypeStruct((NC, NP, D), _f32),
                  jax.ShapeDtypeStruct((NC, NP, D), _f32)],
        mesh=_sc_mesh(),
        compiler_params=pltpu.CompilerParams(use_tc_tiling_on_sc=False),
        scratch_types=[
            pltpu.VMEM((NCH, CH), jnp.int32),
            pltpu.VMEM((NCH, CH), jnp.int32),
            pltpu.VMEM((NBUF, CH, D), _f32),
            pltpu.VMEM_SHARED((NP, D), _f32),
            pltpu.VMEM_SHARED((NP, D), _f32),
            [pltpu.SemaphoreType.DMA] * NBUF,
            [pltpu.SemaphoreType.DMA] * NBUF,
        ],
    )
    def agg2(pa_hbm, pb_hbm, src_hbm, dst_hbm, z_hbm, outa_hbm, outb_hbm,
             src_v, dst_v, rows_v, acc_sh, p_sh, gsems, ssems):
        c = lax.axis_index("c")
        s = lax.axis_index("s")
        wid = s * NC + c
        rows = pl.ds(s * RPT, RPT)
        pltpu.sync_copy(src_hbm.at[wid], src_v)
        pltpu.sync_copy(dst_hbm.at[wid], dst_v)
        for p_hbm, out_hbm in ((pa_hbm, outa_hbm), (pb_hbm, outb_hbm)):
            pltpu.sync_copy(p_hbm.at[rows], p_sh.at[rows])
            pltpu.sync_copy(z_hbm, acc_sh.at[rows])
            plsc.subcore_barrier()
            _edge_pass(p_sh, src_v, dst_v, rows_v, acc_sh, gsems, ssems)
            plsc.subcore_barrier()
            pltpu.sync_copy(acc_sh.at[rows], out_hbm.at[c, rows])
            plsc.subcore_barrier()

    return agg2


def _make_deg():
    @functools.partial(
        pl.kernel,
        out_type=jax.ShapeDtypeStruct((NC, NP, 32), _f32),
        mesh=_sc_mesh(),
        compiler_params=pltpu.CompilerParams(use_tc_tiling_on_sc=False),
        scratch_types=[
            pltpu.VMEM((NCH, CH), jnp.int32),
            pltpu.VMEM((CH, 32), _f32),
            pltpu.VMEM_SHARED((NP, 32), _f32),
            pltpu.SemaphoreType.DMA,
        ],
    )
    def deg(dst_hbm, ones_hbm, z_hbm, out_hbm, dst_v, ones_v, acc_sh, sem):
        c = lax.axis_index("c")
        s = lax.axis_index("s")
        wid = s * NC + c
        pltpu.sync_copy(z_hbm, acc_sh.at[pl.ds(s * RPT, RPT)])
        pltpu.sync_copy(ones_hbm, ones_v)
        pltpu.sync_copy(dst_hbm.at[wid], dst_v)
        plsc.subcore_barrier()

        DEPTH = 8

        def body(j, carry):
            pltpu.async_copy(ones_v, acc_sh.at[dst_v.at[j]], sem, add=True)

            @pl.when(j >= DEPTH)
            def _():
                pltpu.make_async_copy(ones_v, acc_sh.at[dst_v.at[j]],
                                      sem).wait()

            return carry

        lax.fori_loop(0, NCH, body, 0)

        def drain(j, carry):
            pltpu.make_async_copy(ones_v, acc_sh.at[dst_v.at[j]], sem).wait()
            return carry

        lax.fori_loop(0, DEPTH, drain, 0)
        plsc.subcore_barrier()
        pltpu.sync_copy(acc_sh.at[pl.ds(s * RPT, RPT)],
                        out_hbm.at[c, pl.ds(s * RPT, RPT)])

    return deg


def _sf(deg_ref):
    dd = deg_ref[...]
    return lax.rsqrt(dd[0] + dd[1] + 1.0)


def _elu(v):
    return jnp.where(v > 0, v, jnp.exp(v) - 1.0)


def _k_first(x_ref, deg_ref, w_ref, b_ref, o_ref):
    s = _sf(deg_ref)
    m = jnp.dot(x_ref[...], w_ref[...], preferred_element_type=_f32)
    o_ref[...] = s * (m + b_ref[...])


def _k_mid(a_ref, p_ref, deg_ref, w_ref, b_ref, o_ref):
    s = _sf(deg_ref)
    aa = a_ref[...]
    h = _elu(s * (aa[0] + aa[1] + p_ref[...]))
    m = jnp.dot(h, w_ref[...], preferred_element_type=_f32)
    o_ref[...] = s * (m + b_ref[...])


def _k_mid_split(a_ref, p_ref, deg_ref, wa_ref, wb_ref, ba_ref, bb_ref,
                 o1_ref, o2_ref):
    s = _sf(deg_ref)
    aa = a_ref[...]
    h = _elu(s * (aa[0] + aa[1] + p_ref[...]))
    ma = jnp.dot(h, wa_ref[...], preferred_element_type=_f32)
    mb = jnp.dot(h, wb_ref[...], preferred_element_type=_f32)
    o1_ref[...] = s * (ma + ba_ref[...])
    o2_ref[...] = s * (mb + bb_ref[...])


def _k_mid_cat(aa_ref, ab_ref, pa_ref, pb_ref, deg_ref, wa_ref, wb_ref,
               b_ref, o_ref):
    s = _sf(deg_ref)
    va = aa_ref[...]
    vb = ab_ref[...]
    ha = _elu(s * (va[0] + va[1] + pa_ref[...]))
    hb = _elu(s * (vb[0] + vb[1] + pb_ref[...]))
    m = (jnp.dot(ha, wa_ref[...], preferred_element_type=_f32)
         + jnp.dot(hb, wb_ref[...], preferred_element_type=_f32))
    o_ref[...] = s * (m + b_ref[...])


def _k_last(a_ref, p_ref, deg_ref, ones_ref, o_ref):
    s = _sf(deg_ref)
    aa = a_ref[...]
    z = s * (aa[0] + aa[1] + p_ref[...])
    col = lax.broadcasted_iota(jnp.int32, z.shape, 1)
    zm = jnp.where(col % 32 < 28, z, -1e30)
    mx = jnp.max(zm, axis=-1, keepdims=True)
    e = jnp.exp(zm - mx)
    denom = jnp.dot(e, ones_ref[...], preferred_element_type=_f32)
    o_ref[...] = e / denom


def _fold_spec():
    return pl.BlockSpec((BLK4, 128), lambda i: (i, 0))


def _fpart_spec():
    return pl.BlockSpec((NC, BLK4, 128), lambda i: (0, i, 0))


def _full_spec(shape):
    return pl.BlockSpec(shape, lambda i: tuple(0 for _ in shape))


def _tc_first(x, deg, w, b):
    return pl.pallas_call(
        _k_first,
        grid=(GRID,),
        in_specs=[pl.BlockSpec((BLK4, 512), lambda i: (i, 0)), _fpart_spec(),
                  _full_spec((512, 128)), _full_spec((1, 128))],
        out_specs=_fold_spec(),
        out_shape=jax.ShapeDtypeStruct((NP4, 128), _f32),
    )(x, deg, w, b)


def _tc_mid(a, p, deg, w, b):
    return pl.pallas_call(
        _k_mid,
        grid=(GRID,),
        in_specs=[_fpart_spec(), _fold_spec(), _fpart_spec(),
                  _full_spec((128, 128)), _full_spec((1, 128))],
        out_specs=_fold_spec(),
        out_shape=jax.ShapeDtypeStruct((NP4, 128), _f32),
    )(a, p, deg, w, b)


def _tc_mid_split(a, p, deg, wa, wb, ba, bb):
    return pl.pallas_call(
        _k_mid_split,
        grid=(GRID,),
        in_specs=[_fpart_spec(), _fold_spec(), _fpart_spec(),
                  _full_spec((128, 128)), _full_spec((128, 128)),
                  _full_spec((1, 128)), _full_spec((1, 128))],
        out_specs=[_fold_spec(), _fold_spec()],
        out_shape=[jax.ShapeDtypeStruct((NP4, 128), _f32),
                   jax.ShapeDtypeStruct((NP4, 128), _f32)],
    )(a, p, deg, wa, wb, ba, bb)


def _tc_mid_cat(aa, ab, pa, pb, deg, wa, wb, b):
    return pl.pallas_call(
        _k_mid_cat,
        grid=(GRID,),
        in_specs=[_fpart_spec(), _fpart_spec(),
                  _fold_spec(), _fold_spec(), _fpart_spec(),
                  _full_spec((128, 128)), _full_spec((128, 128)),
                  _full_spec((1, 128))],
        out_specs=_fold_spec(),
        out_shape=jax.ShapeDtypeStruct((NP4, 128), _f32),
    )(aa, ab, pa, pb, deg, wa, wb, b)


def _tc_last(a, p, deg, ones_bd):
    return pl.pallas_call(
        _k_last,
        grid=(GRID,),
        in_specs=[_fpart_spec(), _fold_spec(), _fpart_spec(),
                  _full_spec((128, 128))],
        out_specs=_fold_spec(),
        out_shape=jax.ShapeDtypeStruct((NP4, 128), _f32),
    )(a, p, deg, ones_bd)


def kernel(x, edge_index, W1, b1, W2, b2, W3, b3, W4, b4):
    ei = edge_index.astype(jnp.int32)
    pad = jnp.full((EPAD - E,), N, jnp.int32)
    src_r = jnp.concatenate([ei[0], pad]).reshape(NW, NCH, CH)
    dst_r = jnp.concatenate([ei[1], pad]).reshape(NW, NCH, CH)
    x_f = jnp.pad(x, ((0, NP - N), (0, 0))).reshape(NP4, 512)
    w4p = jnp.pad(W4, ((0, 0), (0, 4)))
    b4p = jnp.pad(b4, (0, 4))

    eye4 = jnp.eye(4, dtype=_f32)

    def bd(w):
        return jnp.kron(eye4, w)

    def bf(b):
        return jnp.tile(b, 4).reshape(1, 128)

    ones32 = jnp.ones((CH, 32), _f32)
    z32 = jnp.zeros((RPT, 32), _f32)
    ones_bd = bd(jnp.ones((32, 32), _f32))

    deg = _make_deg()(dst_r, ones32, z32)
    deg_f = deg.reshape(NC, NP4, 128)
    agg32 = _make_agg(32)
    agg32x2 = _make_agg_dual(32)

    def sc(p):
        return p.reshape(NP, 32)

    def tc(a):
        return a.reshape(NC, NP4, 128)

    p1 = _tc_first(x_f, deg_f, bd(W1), bf(b1))
    a1 = agg32(sc(p1), src_r, dst_r, z32)
    p2a, p2b = _tc_mid_split(tc(a1), p1, deg_f, bd(W2[:, :32]),
                             bd(W2[:, 32:]), bf(b2[:32]), bf(b2[32:]))
    a2a, a2b = agg32x2(sc(p2a), sc(p2b), src_r, dst_r, z32)
    p3 = _tc_mid_cat(tc(a2a), tc(a2b), p2a, p2b, deg_f,
                     bd(W3[:32, :]), bd(W3[32:, :]), bf(b3))
    a3 = agg32(sc(p3), src_r, dst_r, z32)
    p4 = _tc_mid(tc(a3), p3, deg_f, bd(w4p), bf(b4p))
    a4 = agg32(sc(p4), src_r, dst_r, z32)
    out = _tc_last(tc(a4), p4, deg_f, ones_bd)
    return out.reshape(NP, 32)[:N, :28]


# speedup vs baseline: 46.2767x; 1.0014x over previous
import functools

import jax
import jax.numpy as jnp
from jax import lax
from jax.experimental import pallas as pl
from jax.experimental.pallas import tpu as pltpu
from jax.experimental.pallas import tpu_sc as plsc

N = 10000
NP = 10240
E = 320000
NC, NS, LANES = 2, 16, 16
NW = NC * NS
CH = 128
NCH = 80
EPAD = NW * NCH * CH
RPT = NP // NS
NBUF = 8
GAH = 3
BLK = 1024
GRID = NP // BLK
BLK4 = BLK // 4
NP4 = NP // 4

_f32 = jnp.float32


def _sc_mesh():
    return plsc.VectorSubcoreMesh(
        core_axis_name="c", subcore_axis_name="s",
        num_cores=NC, num_subcores=NS)


def _edge_pass(p_src, src_v, dst_v, rows_v, acc_sh, gsems, ssems):
    def gather(chunk, buf):
        pltpu.async_copy(p_src.at[src_v.at[chunk]], rows_v.at[buf],
                         gsems[buf])

    def gwait(chunk, buf):
        pltpu.make_async_copy(p_src.at[src_v.at[chunk]], rows_v.at[buf],
                              gsems[buf]).wait()

    def scatter(chunk, buf):
        pltpu.async_copy(rows_v.at[buf], acc_sh.at[dst_v.at[chunk]],
                         ssems[buf], add=True)

    def swait(chunk, buf):
        pltpu.make_async_copy(rows_v.at[buf], acc_sh.at[dst_v.at[chunk]],
                              ssems[buf]).wait()

    for k in range(GAH):
        gather(k, k)

    def body(g, carry):
        for b in range(NBUF):
            j = g * NBUF + b
            gwait(j, b)
            scatter(j, b)
            nxt = j + GAH
            nb = (b + GAH) % NBUF

            @pl.when(nxt < NCH)
            def _():
                @pl.when(nxt >= NBUF)
                def _():
                    swait(nxt - NBUF, nb)

                gather(nxt, nb)

        return carry

    lax.fori_loop(0, NCH // NBUF, body, 0)
    for b in range(NBUF):
        swait(NCH - NBUF + b, b)


def _make_agg(D):
    @functools.partial(
        pl.kernel,
        out_type=jax.ShapeDtypeStruct((NC, NP, D), _f32),
        mesh=_sc_mesh(),
        compiler_params=pltpu.CompilerParams(use_tc_tiling_on_sc=False),
        scratch_types=[
            pltpu.VMEM((NCH, CH), jnp.int32),
            pltpu.VMEM((NCH, CH), jnp.int32),
            pltpu.VMEM((NBUF, CH, D), _f32),
            pltpu.VMEM_SHARED((NP, D), _f32),
            pltpu.VMEM_SHARED((NP, D), _f32),
            [pltpu.SemaphoreType.DMA] * NBUF,
            [pltpu.SemaphoreType.DMA] * NBUF,
        ],
    )
    def agg(p_hbm, src_hbm, dst_hbm, z_hbm, out_hbm,
            src_v, dst_v, rows_v, acc_sh, p_sh, gsems, ssems):
        c = lax.axis_index("c")
        s = lax.axis_index("s")
        wid = s * NC + c
        rows = pl.ds(s * RPT, RPT)
        pltpu.sync_copy(p_hbm.at[rows], p_sh.at[rows])
        pltpu.sync_copy(z_hbm, acc_sh.at[rows])
        pltpu.sync_copy(src_hbm.at[wid], src_v)
        pltpu.sync_copy(dst_hbm.at[wid], dst_v)
        plsc.subcore_barrier()
        _edge_pass(p_sh, src_v, dst_v, rows_v, acc_sh, gsems, ssems)
        plsc.subcore_barrier()
        pltpu.sync_copy(acc_sh.at[rows], out_hbm.at[c, rows])

    return agg


def _make_agg_dual(D):
    @functools.partial(
        pl.kernel,
        out_type=[jax.ShapeDtypeStruct((NC, NP, D), _f32),
                  jax.ShapeDtypeStruct((NC, NP, D), _f32)],
        mesh=_sc_mesh(),
        compiler_params=pltpu.CompilerParams(use_tc_tiling_on_sc=False),
        scratch_types=[
            pltpu.VMEM((NCH, CH), jnp.int32),
            pltpu.VMEM((NCH, CH), jnp.int32),
            pltpu.VMEM((NBUF, CH, D), _f32),
            pltpu.VMEM_SHARED((NP, D), _f32),
            pltpu.VMEM_SHARED((NP, D), _f32),
            [pltpu.SemaphoreType.DMA] * NBUF,
            [pltpu.SemaphoreType.DMA] * NBUF,
        ],
    )
    def agg2(pa_hbm, pb_hbm, src_hbm, dst_hbm, z_hbm, outa_hbm, outb_hbm,
             src_v, dst_v, rows_v, acc_sh, p_sh, gsems, ssems):
        c = lax.axis_index("c")
        s = lax.axis_index("s")
        wid = s * NC + c
        rows = pl.ds(s * RPT, RPT)
        pltpu.sync_copy(src_hbm.at[wid], src_v)
        pltpu.sync_copy(dst_hbm.at[wid], dst_v)
        for p_hbm, out_hbm in ((pa_hbm, outa_hbm), (pb_hbm, outb_hbm)):
            pltpu.sync_copy(p_hbm.at[rows], p_sh.at[rows])
            pltpu.sync_copy(z_hbm, acc_sh.at[rows])
            plsc.subcore_barrier()
            _edge_pass(p_sh, src_v, dst_v, rows_v, acc_sh, gsems, ssems)
            plsc.subcore_barrier()
            pltpu.sync_copy(acc_sh.at[rows], out_hbm.at[c, rows])
            plsc.subcore_barrier()

    return agg2


def _make_deg():
    @functools.partial(
        pl.kernel,
        out_type=jax.ShapeDtypeStruct((NC, NP, 32), _f32),
        mesh=_sc_mesh(),
        compiler_params=pltpu.CompilerParams(use_tc_tiling_on_sc=False),
        scratch_types=[
            pltpu.VMEM((NCH, CH), jnp.int32),
            pltpu.VMEM((CH, 32), _f32),
            pltpu.VMEM_SHARED((NP, 32), _f32),
            pltpu.SemaphoreType.DMA,
        ],
    )
    def deg(dst_hbm, ones_hbm, z_hbm, out_hbm, dst_v, ones_v, acc_sh, sem):
        c = lax.axis_index("c")
        s = lax.axis_index("s")
        wid = s * NC + c
        pltpu.sync_copy(z_hbm, acc_sh.at[pl.ds(s * RPT, RPT)])
        pltpu.sync_copy(ones_hbm, ones_v)
        pltpu.sync_copy(dst_hbm.at[wid], dst_v)
        plsc.subcore_barrier()

        DEPTH = 8

        def body(j, carry):
            pltpu.async_copy(ones_v, acc_sh.at[dst_v.at[j]], sem, add=True)

            @pl.when(j >= DEPTH)
            def _():
                pltpu.make_async_copy(ones_v, acc_sh.at[dst_v.at[j]],
                                      sem).wait()

            return carry

        lax.fori_loop(0, NCH, body, 0)

        def drain(j, carry):
            pltpu.make_async_copy(ones_v, acc_sh.at[dst_v.at[j]], sem).wait()
            return carry

        lax.fori_loop(0, DEPTH, drain, 0)
        plsc.subcore_barrier()
        pltpu.sync_copy(acc_sh.at[pl.ds(s * RPT, RPT)],
                        out_hbm.at[c, pl.ds(s * RPT, RPT)])

    return deg


def _sf(deg_ref):
    dd = deg_ref[...]
    return lax.rsqrt(dd[0] + dd[1] + 1.0)


def _elu(v):
    return jnp.where(v > 0, v, jnp.exp(v) - 1.0)


def _k_first(x_ref, deg_ref, w_ref, b_ref, o_ref):
    s = _sf(deg_ref)
    m = jnp.dot(x_ref[...], w_ref[...], preferred_element_type=_f32)
    o_ref[...] = s * (m + b_ref[...])


def _k_mid(a_ref, p_ref, deg_ref, w_ref, b_ref, o_ref):
    s = _sf(deg_ref)
    aa = a_ref[...]
    h = _elu(s * (aa[0] + aa[1] + p_ref[...]))
    m = jnp.dot(h, w_ref[...], preferred_element_type=_f32)
    o_ref[...] = s * (m + b_ref[...])


def _k_mid_split(a_ref, p_ref, deg_ref, wa_ref, wb_ref, ba_ref, bb_ref,
                 o1_ref, o2_ref):
    s = _sf(deg_ref)
    aa = a_ref[...]
    h = _elu(s * (aa[0] + aa[1] + p_ref[...]))
    ma = jnp.dot(h, wa_ref[...], preferred_element_type=_f32)
    mb = jnp.dot(h, wb_ref[...], preferred_element_type=_f32)
    o1_ref[...] = s * (ma + ba_ref[...])
    o2_ref[...] = s * (mb + bb_ref[...])


def _k_mid_cat(aa_ref, ab_ref, pa_ref, pb_ref, deg_ref, wa_ref, wb_ref,
               b_ref, o_ref):
    s = _sf(deg_ref)
    va = aa_ref[...]
    vb = ab_ref[...]
    ha = _elu(s * (va[0] + va[1] + pa_ref[...]))
    hb = _elu(s * (vb[0] + vb[1] + pb_ref[...]))
    m = (jnp.dot(ha, wa_ref[...], preferred_element_type=_f32)
         + jnp.dot(hb, wb_ref[...], preferred_element_type=_f32))
    o_ref[...] = s * (m + b_ref[...])


def _k_last(a_ref, p_ref, deg_ref, ones_ref, o_ref):
    s = _sf(deg_ref)
    aa = a_ref[...]
    z = s * (aa[0] + aa[1] + p_ref[...])
    col = lax.broadcasted_iota(jnp.int32, z.shape, 1)
    zm = jnp.where(col % 32 < 28, z, -1e30)
    mx = jnp.max(zm, axis=-1, keepdims=True)
    e = jnp.exp(zm - mx)
    denom = jnp.dot(e, ones_ref[...], preferred_element_type=_f32)
    o_ref[...] = e / denom


def _fold_spec():
    return pl.BlockSpec((BLK4, 128), lambda i: (i, 0))


def _fpart_spec():
    return pl.BlockSpec((NC, BLK4, 128), lambda i: (0, i, 0))


def _full_spec(shape):
    return pl.BlockSpec(shape, lambda i: tuple(0 for _ in shape))


def _tc_first(x, deg, w, b):
    return pl.pallas_call(
        _k_first,
        grid=(GRID,),
        in_specs=[pl.BlockSpec((BLK4, 512), lambda i: (i, 0)), _fpart_spec(),
                  _full_spec((512, 128)), _full_spec((1, 128))],
        out_specs=_fold_spec(),
        out_shape=jax.ShapeDtypeStruct((NP4, 128), _f32),
    )(x, deg, w, b)


def _tc_mid(a, p, deg, w, b):
    return pl.pallas_call(
        _k_mid,
        grid=(GRID,),
        in_specs=[_fpart_spec(), _fold_spec(), _fpart_spec(),
                  _full_spec((128, 128)), _full_spec((1, 128))],
        out_specs=_fold_spec(),
        out_shape=jax.ShapeDtypeStruct((NP4, 128), _f32),
    )(a, p, deg, w, b)


def _tc_mid_split(a, p, deg, wa, wb, ba, bb):
    return pl.pallas_call(
        _k_mid_split,
        grid=(GRID,),
        in_specs=[_fpart_spec(), _fold_spec(), _fpart_spec(),
                  _full_spec((128, 128)), _full_spec((128, 128)),
                  _full_spec((1, 128)), _full_spec((1, 128))],
        out_specs=[_fold_spec(), _fold_spec()],
        out_shape=[jax.ShapeDtypeStruct((NP4, 128), _f32),
                   jax.ShapeDtypeStruct((NP4, 128), _f32)],
    )(a, p, deg, wa, wb, ba, bb)


def _tc_mid_cat(aa, ab, pa, pb, deg, wa, wb, b):
    return pl.pallas_call(
        _k_mid_cat,
        grid=(GRID,),
        in_specs=[_fpart_spec(), _fpart_spec(),
                  _fold_spec(), _fold_spec(), _fpart_spec(),
                  _full_spec((128, 128)), _full_spec((128, 128)),
                  _full_spec((1, 128))],
        out_specs=_fold_spec(),
        out_shape=jax.ShapeDtypeStruct((NP4, 128), _f32),
    )(aa, ab, pa, pb, deg, wa, wb, b)


def _tc_last(a, p, deg, ones_bd):
    return pl.pallas_call(
        _k_last,
        grid=(GRID,),
        in_specs=[_fpart_spec(), _fold_spec(), _fpart_spec(),
                  _full_spec((128, 128))],
        out_specs=_fold_spec(),
        out_shape=jax.ShapeDtypeStruct((NP4, 128), _f32),
    )(a, p, deg, ones_bd)


def kernel(x, edge_index, W1, b1, W2, b2, W3, b3, W4, b4):
    ei = edge_index.astype(jnp.int32)
    pad = jnp.full((EPAD - E,), N, jnp.int32)
    src_r = jnp.concatenate([ei[0], pad]).reshape(NW, NCH, CH)
    dst_r = jnp.concatenate([ei[1], pad]).reshape(NW, NCH, CH)
    x_f = jnp.pad(x, ((0, NP - N), (0, 0))).reshape(NP4, 512)
    w4p = jnp.pad(W4, ((0, 0), (0, 4)))
    b4p = jnp.pad(b4, (0, 4))

    eye4 = jnp.eye(4, dtype=_f32)

    def bd(w):
        return jnp.kron(eye4, w)

    def bf(b):
        return jnp.tile(b, 4).reshape(1, 128)

    ones32 = jnp.ones((CH, 32), _f32)
    z32 = jnp.zeros((RPT, 32), _f32)
    ones_bd = bd(jnp.ones((32, 32), _f32))

    deg = _make_deg()(dst_r, ones32, z32)
    deg_f = deg.reshape(NC, NP4, 128)
    agg32 = _make_agg(32)
    agg32x2 = _make_agg_dual(32)

    def sc(p):
        return p.reshape(NP, 32)

    def tc(a):
        return a.reshape(NC, NP4, 128)

    p1 = _tc_first(x_f, deg_f, bd(W1), bf(b1))
    a1 = agg32(sc(p1), src_r, dst_r, z32)
    p2a, p2b = _tc_mid_split(tc(a1), p1, deg_f, bd(W2[:, :32]),
                             bd(W2[:, 32:]), bf(b2[:32]), bf(b2[32:]))
    a2a, a2b = agg32x2(sc(p2a), sc(p2b), src_r, dst_r, z32)
    p3 = _tc_mid_cat(tc(a2a), tc(a2b), p2a, p2b, deg_f,
                     bd(W3[:32, :]), bd(W3[32:, :]), bf(b3))
    a3 = agg32(sc(p3), src_r, dst_r, z32)
    p4 = _tc_mid(tc(a3), p3, deg_f, bd(w4p), bf(b4p))
    a4 = agg32(sc(p4), src_r, dst_r, z32)
    out = _tc_last(tc(a4), p4, deg_f, ones_bd)
    return out.reshape(NP, 32)[:N, :28]
